# baseline (speedup 1.0000x reference)
import functools

import jax
import jax.numpy as jnp
from jax import lax
from jax.experimental import pallas as pl
from jax.experimental.pallas import tpu as pltpu
from jax.experimental.pallas import tpu_sc as plsc

NL_DIM = 128
TP_DIM = 64
POS_DIM = 64
OUT_DIM = NL_DIM + TP_DIM + POS_DIM
NC, NS = 2, 16
NW = NC * NS
NSLOT = 2


def _make_kernel(B, L):
    bpw = B // NW
    mesh = plsc.VectorSubcoreMesh(core_axis_name="c", subcore_axis_name="s",
                                  num_cores=NC, num_subcores=NS)

    @functools.partial(
        pl.kernel,
        out_type=jax.ShapeDtypeStruct((L, B, OUT_DIM), jnp.float32),
        mesh=mesh,
        scratch_types=[
            pltpu.VMEM((L, bpw), jnp.int32),
            pltpu.VMEM((L, bpw), jnp.int32),
            [pltpu.VMEM((bpw, NL_DIM), jnp.float32) for _ in range(NSLOT)],
            [pltpu.VMEM((bpw, NL_DIM), jnp.float32) for _ in range(NSLOT)],
            pltpu.VMEM_SHARED((6000, TP_DIM + POS_DIM), jnp.float32),
            [pltpu.SemaphoreType.DMA for _ in range(NSLOT)],
            [pltpu.SemaphoreType.DMA for _ in range(NSLOT)],
            [pltpu.SemaphoreType.DMA for _ in range(NSLOT)],
            [pltpu.SemaphoreType.DMA for _ in range(NSLOT)],
        ],
    )
    def k(nl_hbm, cx_hbm, wombat_hbm, wnl_hbm, wcross_hbm,
          out_hbm, nl_idx, cx_idx, accs, tpps, wcross_sp,
          sem_a, sem_g, sem_c, sem_w):
        wid = lax.axis_index("s") * NC + lax.axis_index("c")
        wb = wid * bpw
        pltpu.sync_copy(nl_hbm.at[:, pl.ds(wb, bpw)], nl_idx)
        pltpu.sync_copy(cx_hbm.at[:, pl.ds(wb, bpw)], cx_idx)

        @pl.when(lax.axis_index("s") == 0)
        def _():
            pltpu.sync_copy(wcross_hbm, wcross_sp)
        plsc.subcore_barrier()

        def fire_loads(s, l):
            pltpu.async_copy(wombat_hbm.at[l, pl.ds(wb, bpw), :], accs[s],
                             sem_a[s])
            pltpu.async_copy(wcross_sp.at[cx_idx.at[l]], tpps[s], sem_c[s])

        def wait_wombat_fire_adds(s, l):
            pltpu.make_async_copy(wombat_hbm.at[l, pl.ds(wb, bpw), :],
                                  accs[s], sem_a[s]).wait()
            pltpu.async_copy(wnl_hbm.at[nl_idx.at[l]], accs[s], sem_g[s],
                             add=True)

        def wait_fire_writes(s, l):
            pltpu.make_async_copy(wnl_hbm.at[nl_idx.at[l]], accs[s],
                                  sem_g[s]).wait()
            pltpu.make_async_copy(wcross_sp.at[cx_idx.at[l]], tpps[s],
                                  sem_c[s]).wait()
            pltpu.async_copy(
                accs[s], out_hbm.at[l, pl.ds(wb, bpw), pl.ds(0, NL_DIM)],
                sem_w[s])
            pltpu.async_copy(
                tpps[s], out_hbm.at[l, pl.ds(wb, bpw), pl.ds(NL_DIM, NL_DIM)],
                sem_w[s])

        def wait_writes(s, l):
            pltpu.make_async_copy(
                accs[s], out_hbm.at[l, pl.ds(wb, bpw), pl.ds(0, NL_DIM)],
                sem_w[s]).wait()
            pltpu.make_async_copy(
                tpps[s], out_hbm.at[l, pl.ds(wb, bpw), pl.ds(NL_DIM, NL_DIM)],
                sem_w[s]).wait()

        steps = -(-L // NSLOT)

        def body(i, carry):
            for s in range(NSLOT):
                l = NSLOT * i + s

                @pl.when((i > 0) & (l < L))
                def _():
                    wait_writes(s, l)

                @pl.when(l < L)
                def _():
                    fire_loads(s, l)
            for s in range(NSLOT):
                l = NSLOT * i + s

                @pl.when(l < L)
                def _():
                    wait_wombat_fire_adds(s, l)
            for s in range(NSLOT):
                l = NSLOT * i + s

                @pl.when(l < L)
                def _():
                    wait_fire_writes(s, l)
            return carry

        lax.fori_loop(0, steps, body, 0)
        for s in range(NSLOT):
            if NSLOT * (steps - 1) + s < L:
                wait_writes(s, NSLOT * (steps - 1) + s)
            else:
                wait_writes(s, NSLOT * (steps - 2) + s)

    return k


def kernel(nl_tensor, tp_tensor, pos_tensor, wombat_tensor, W_nl, W_tp, W_pos):
    B, L = nl_tensor.shape
    pos_vocab = W_pos.shape[0]
    w_cross = jnp.concatenate(
        [jnp.broadcast_to(W_tp[:, None, :], (W_tp.shape[0], pos_vocab, TP_DIM)),
         jnp.broadcast_to(W_pos[None, :, :], (W_tp.shape[0], pos_vocab, POS_DIM))],
        axis=-1).reshape(W_tp.shape[0] * pos_vocab, TP_DIM + POS_DIM)
    cx_tensor = tp_tensor * pos_vocab + pos_tensor
    k = _make_kernel(B, L)
    out_t = k(nl_tensor.T, cx_tensor.T, wombat_tensor.transpose(1, 0, 2),
              W_nl, w_cross)
    return out_t.transpose(1, 0, 2)


# speedup vs baseline: 1.1066x; 1.1066x over previous
import functools

import jax
import jax.numpy as jnp
from jax import lax
from jax.experimental import pallas as pl
from jax.experimental.pallas import tpu as pltpu
from jax.experimental.pallas import tpu_sc as plsc

NL_DIM = 128
TP_DIM = 64
POS_DIM = 64
OUT_DIM = NL_DIM + TP_DIM + POS_DIM
NC, NS = 2, 16
NW = NC * NS
NSLOT = 2


def _make_kernel(B, L):
    bpw = B // NW
    mesh = plsc.VectorSubcoreMesh(core_axis_name="c", subcore_axis_name="s",
                                  num_cores=NC, num_subcores=NS)

    @functools.partial(
        pl.kernel,
        out_type=jax.ShapeDtypeStruct((L, B, OUT_DIM), jnp.float32),
        mesh=mesh,
        scratch_types=[
            pltpu.VMEM((L, bpw), jnp.int32),
            pltpu.VMEM((L, bpw), jnp.int32),
            [pltpu.VMEM((bpw, NL_DIM), jnp.float32) for _ in range(NSLOT)],
            [pltpu.VMEM((bpw, NL_DIM), jnp.float32) for _ in range(NSLOT)],
            pltpu.VMEM_SHARED((6000, TP_DIM + POS_DIM), jnp.float32),
            [pltpu.SemaphoreType.DMA for _ in range(NSLOT)],
            [pltpu.SemaphoreType.DMA for _ in range(NSLOT)],
            [pltpu.SemaphoreType.DMA for _ in range(NSLOT)],
            [pltpu.SemaphoreType.DMA for _ in range(NSLOT)],
        ],
    )
    def k(nl_hbm, cx_hbm, wombat_hbm, wnl_hbm, wcross_hbm,
          out_hbm, nl_idx, cx_idx, accs, tpps, wcross_sp,
          sem_a, sem_g, sem_c, sem_w):
        wid = lax.axis_index("s") * NC + lax.axis_index("c")
        wb = wid * bpw
        pltpu.sync_copy(nl_hbm.at[:, pl.ds(wb, bpw)], nl_idx)
        pltpu.sync_copy(cx_hbm.at[:, pl.ds(wb, bpw)], cx_idx)

        @pl.when(lax.axis_index("s") == 0)
        def _():
            pltpu.sync_copy(wcross_hbm, wcross_sp)
        plsc.subcore_barrier()

        def fire_loads(s, l):
            pltpu.async_copy(wombat_hbm.at[l, pl.ds(wb, bpw), :], accs[s],
                             sem_a[s])
            pltpu.async_copy(wcross_sp.at[cx_idx.at[l]], tpps[s], sem_c[s])

        def wait_wombat_fire_adds(s, l):
            pltpu.make_async_copy(wombat_hbm.at[l, pl.ds(wb, bpw), :],
                                  accs[s], sem_a[s]).wait()
            pltpu.async_copy(wnl_hbm.at[nl_idx.at[l]], accs[s], sem_g[s],
                             add=True)

        def wait_fire_writes(s, l):
            pltpu.make_async_copy(wcross_sp.at[cx_idx.at[l]], tpps[s],
                                  sem_c[s]).wait()
            pltpu.async_copy(
                tpps[s], out_hbm.at[l, pl.ds(wb, bpw), pl.ds(NL_DIM, NL_DIM)],
                sem_w[s])
            pltpu.make_async_copy(wnl_hbm.at[nl_idx.at[l]], accs[s],
                                  sem_g[s]).wait()
            pltpu.async_copy(
                accs[s], out_hbm.at[l, pl.ds(wb, bpw), pl.ds(0, NL_DIM)],
                sem_w[s])

        def wait_writes(s, l):
            pltpu.make_async_copy(
                accs[s], out_hbm.at[l, pl.ds(wb, bpw), pl.ds(0, NL_DIM)],
                sem_w[s]).wait()
            pltpu.make_async_copy(
                tpps[s], out_hbm.at[l, pl.ds(wb, bpw), pl.ds(NL_DIM, NL_DIM)],
                sem_w[s]).wait()

        steps = -(-L // NSLOT)

        def body(i, carry):
            for s in range(NSLOT):
                l = NSLOT * i + s

                @pl.when((i > 0) & (l < L))
                def _():
                    wait_writes(s, l)

                @pl.when(l < L)
                def _():
                    fire_loads(s, l)
            for s in range(NSLOT):
                l = NSLOT * i + s

                @pl.when(l < L)
                def _():
                    wait_wombat_fire_adds(s, l)
            for s in range(NSLOT):
                l = NSLOT * i + s

                @pl.when(l < L)
                def _():
                    wait_fire_writes(s, l)
            return carry

        lax.fori_loop(0, steps, body, 0)
        for s in range(NSLOT):
            if NSLOT * (steps - 1) + s < L:
                wait_writes(s, NSLOT * (steps - 1) + s)
            else:
                wait_writes(s, NSLOT * (steps - 2) + s)

    return k


def kernel(nl_tensor, tp_tensor, pos_tensor, wombat_tensor, W_nl, W_tp, W_pos):
    B, L = nl_tensor.shape
    pos_vocab = W_pos.shape[0]
    w_cross = jnp.concatenate(
        [jnp.broadcast_to(W_tp[:, None, :], (W_tp.shape[0], pos_vocab, TP_DIM)),
         jnp.broadcast_to(W_pos[None, :, :], (W_tp.shape[0], pos_vocab, POS_DIM))],
        axis=-1).reshape(W_tp.shape[0] * pos_vocab, TP_DIM + POS_DIM)
    cx_tensor = tp_tensor * pos_vocab + pos_tensor
    k = _make_kernel(B, L)
    out_t = k(nl_tensor.T, cx_tensor.T, wombat_tensor.transpose(1, 0, 2),
              W_nl, w_cross)
    return out_t.transpose(1, 0, 2)


# speedup vs baseline: 1.1156x; 1.0081x over previous
import functools

import jax
import jax.numpy as jnp
from jax import lax
from jax.experimental import pallas as pl
from jax.experimental.pallas import tpu as pltpu
from jax.experimental.pallas import tpu_sc as plsc

NL_DIM = 128
TP_DIM = 64
POS_DIM = 64
OUT_DIM = NL_DIM + TP_DIM + POS_DIM
NC, NS = 2, 16
NW = NC * NS
NSLOT = 4
HALF = 2


def _make_kernel(B, L):
    bpw = B // NW
    mesh = plsc.VectorSubcoreMesh(core_axis_name="c", subcore_axis_name="s",
                                  num_cores=NC, num_subcores=NS)

    @functools.partial(
        pl.kernel,
        out_type=jax.ShapeDtypeStruct((L, B, OUT_DIM), jnp.float32),
        mesh=mesh,
        scratch_types=[
            pltpu.VMEM((L, bpw), jnp.int32),
            pltpu.VMEM((L, bpw), jnp.int32),
            [pltpu.VMEM((bpw // HALF, NL_DIM), jnp.float32)
             for _ in range(NSLOT)],
            [pltpu.VMEM((bpw // HALF, NL_DIM), jnp.float32)
             for _ in range(NSLOT)],
            pltpu.VMEM_SHARED((6000, TP_DIM + POS_DIM), jnp.float32),
            [pltpu.SemaphoreType.DMA for _ in range(NSLOT)],
            [pltpu.SemaphoreType.DMA for _ in range(NSLOT)],
            [pltpu.SemaphoreType.DMA for _ in range(NSLOT)],
            [pltpu.SemaphoreType.DMA for _ in range(NSLOT)],
        ],
    )
    def k(nl_hbm, cx_hbm, wombat_hbm, wnl_hbm, wcross_hbm,
          out_hbm, nl_idx, cx_idx, accs, tpps, wcross_sp,
          sem_a, sem_g, sem_c, sem_w):
        wid = lax.axis_index("s") * NC + lax.axis_index("c")
        wb = wid * bpw
        pltpu.sync_copy(nl_hbm.at[:, pl.ds(wb, bpw)], nl_idx)
        pltpu.sync_copy(cx_hbm.at[:, pl.ds(wb, bpw)], cx_idx)

        @pl.when(lax.axis_index("s") == 0)
        def _():
            pltpu.sync_copy(wcross_hbm, wcross_sp)
        plsc.subcore_barrier()

        hb = bpw // HALF

        def fire_loads(s, q):
            l, h = q // HALF, q % HALF
            b0 = wb + h * hb
            pltpu.async_copy(wombat_hbm.at[l, pl.ds(b0, hb), :], accs[s],
                             sem_a[s])
            pltpu.async_copy(wcross_sp.at[cx_idx.at[l, pl.ds(h * hb, hb)]],
                             tpps[s], sem_c[s])

        def wait_wombat_fire_adds(s, q):
            l, h = q // HALF, q % HALF
            b0 = wb + h * hb
            pltpu.make_async_copy(wombat_hbm.at[l, pl.ds(b0, hb), :],
                                  accs[s], sem_a[s]).wait()
            pltpu.async_copy(wnl_hbm.at[nl_idx.at[l, pl.ds(h * hb, hb)]],
                             accs[s], sem_g[s], add=True)

        def wait_fire_writes(s, q):
            l, h = q // HALF, q % HALF
            b0 = wb + h * hb
            pltpu.make_async_copy(wcross_sp.at[cx_idx.at[l, pl.ds(h * hb, hb)]],
                                  tpps[s], sem_c[s]).wait()
            pltpu.async_copy(
                tpps[s], out_hbm.at[l, pl.ds(b0, hb), pl.ds(NL_DIM, NL_DIM)],
                sem_w[s])
            pltpu.make_async_copy(wnl_hbm.at[nl_idx.at[l, pl.ds(h * hb, hb)]],
                                  accs[s], sem_g[s]).wait()
            pltpu.async_copy(
                accs[s], out_hbm.at[l, pl.ds(b0, hb), pl.ds(0, NL_DIM)],
                sem_w[s])

        def wait_writes(s, q):
            l, h = q // HALF, q % HALF
            b0 = wb + h * hb
            pltpu.make_async_copy(
                accs[s], out_hbm.at[l, pl.ds(b0, hb), pl.ds(0, NL_DIM)],
                sem_w[s]).wait()
            pltpu.make_async_copy(
                tpps[s], out_hbm.at[l, pl.ds(b0, hb), pl.ds(NL_DIM, NL_DIM)],
                sem_w[s]).wait()

        nq = L * HALF
        steps = nq // NSLOT

        def body(i, carry):
            for s in range(NSLOT):
                q = NSLOT * i + s

                @pl.when(i > 0)
                def _():
                    wait_writes(s, q)
                fire_loads(s, q)
            for s in range(NSLOT):
                wait_wombat_fire_adds(s, NSLOT * i + s)
            for s in range(NSLOT):
                wait_fire_writes(s, NSLOT * i + s)
            return carry

        lax.fori_loop(0, steps, body, 0)
        for s in range(NSLOT):
            wait_writes(s, NSLOT * (steps - 1) + s)

    return k


def kernel(nl_tensor, tp_tensor, pos_tensor, wombat_tensor, W_nl, W_tp, W_pos):
    B, L = nl_tensor.shape
    pos_vocab = W_pos.shape[0]
    w_cross = jnp.concatenate(
        [jnp.broadcast_to(W_tp[:, None, :], (W_tp.shape[0], pos_vocab, TP_DIM)),
         jnp.broadcast_to(W_pos[None, :, :], (W_tp.shape[0], pos_vocab, POS_DIM))],
        axis=-1).reshape(W_tp.shape[0] * pos_vocab, TP_DIM + POS_DIM)
    cx_tensor = tp_tensor * pos_vocab + pos_tensor
    k = _make_kernel(B, L)
    out_t = k(nl_tensor.T, cx_tensor.T, wombat_tensor.transpose(1, 0, 2),
              W_nl, w_cross)
    return out_t.transpose(1, 0, 2)
